# baseline (speedup 1.0000x reference)
import functools

import jax
import jax.numpy as jnp
from jax import lax
from jax.experimental import pallas as pl
from jax.experimental.pallas import tpu as pltpu
from jax.experimental.pallas import tpu_sc as plsc

N = 10000
E = 160000
C = 128

NP = 10240
NPA = 10112
EP = 163840
TILES = 32
E_TILE = EP // TILES
KCH = 64
NCHUNK = E_TILE // KCH
IB = 4
NBLK = NCHUNK // IB
NBLK_SLOW = 12
NBLK_FAST = 28
ROWS_PER_TILE = NPA // 16


def _mm_body(x_ref, w_ref, a2_ref, xp_ref, alar_ref):
    xb = x_ref[...]
    xp = jnp.dot(xb, w_ref[...], preferred_element_type=jnp.float32)
    xp_ref[...] = xp
    alar_ref[...] = jnp.dot(xp, a2_ref[...], preferred_element_type=jnp.float32)


def _project(x_pad, W, att2):
    return pl.pallas_call(
        _mm_body,
        grid=(NP // 512,),
        in_specs=[
            pl.BlockSpec((512, C), lambda b: (b, 0)),
            pl.BlockSpec((C, C), lambda b: (0, 0)),
            pl.BlockSpec((C, 2), lambda b: (0, 0)),
        ],
        out_specs=[
            pl.BlockSpec((512, C), lambda b: (b, 0)),
            pl.BlockSpec((512, 2), lambda b: (b, 0)),
        ],
        out_shape=[
            jax.ShapeDtypeStruct((NP, C), jnp.float32),
            jax.ShapeDtypeStruct((NP, 2), jnp.float32),
        ],
    )(x_pad, W, att2)


def _sc_body(alar_hbm, row_hbm, col_hbm, xp_hbm, u_hbm, s_hbm,
             alar_v, idxrA, idxrB, idxcA, idxcB, rows0, rows1, pA, pB,
             sidx_v, zrow_v, u_sh, s_sh, gsem, isem):
    idxr = (idxrA, idxrB)
    idxc = (idxcA, idxcB)
    rows = (rows0, rows1)
    pb = (pA, pB)

    c = lax.axis_index("c")
    s = lax.axis_index("s")
    rbase = s * ROWS_PER_TILE
    nblk = jnp.where(c == 0, NBLK_SLOW, NBLK_FAST)
    cbase = jnp.where(c == 0, s * (NBLK_SLOW * IB),
                      16 * (NBLK_SLOW * IB) + s * (NBLK_FAST * IB))

    def blk(j):
        return cbase + j * IB

    zero16 = jnp.zeros((16,), jnp.float32)

    def _zrows(i, _):
        r = i // 8
        j = i - r * 8
        rows0[r, pl.ds(j * 16, 16)] = zero16
        return 0
    lax.fori_loop(0, KCH * 8, _zrows, 0)

    def _zline(i, _):
        zrow_v[pl.ds(i * 16, 16)] = zero16
        return 0
    lax.fori_loop(0, 640 // 16, _zline, 0)

    sbase = s * 640

    for k in range(ROWS_PER_TILE // KCH):
        pltpu.sync_copy(rows0, u_sh.at[pl.ds(rbase + k * KCH, KCH)])
    rem = ROWS_PER_TILE - (ROWS_PER_TILE // KCH) * KCH
    pltpu.sync_copy(
        rows0.at[pl.ds(0, rem)],
        u_sh.at[pl.ds(rbase + ROWS_PER_TILE - rem, rem)])
    pltpu.sync_copy(zrow_v, s_sh.at[pl.ds(sbase, 640)])

    pltpu.sync_copy(alar_hbm, alar_v)

    plsc.subcore_barrier()

    pltpu.sync_copy(row_hbm.at[pl.ds(blk(0), IB)], idxrA)
    pltpu.sync_copy(col_hbm.at[pl.ds(blk(0), IB)], idxcA)
    pltpu.async_copy(row_hbm.at[pl.ds(blk(1), IB)], idxrB, isem)
    pltpu.async_copy(col_hbm.at[pl.ds(blk(1), IB)], idxcB, isem)
    pltpu.async_copy(xp_hbm.at[idxcA.at[0]], rows0, gsem)

    def _super(g2, _):
        for jj in (0, 1):
            j = 2 * g2 + jj
            X = jj
            Y = 1 - jj
            for b in range(IB):
                rb = b % 2
                nrb = 1 - rb
                if b == IB - 1:
                    pltpu.make_async_copy(
                        row_hbm.at[pl.ds(blk(j + 1), IB)],
                        idxr[Y], isem).wait()
                    pltpu.make_async_copy(
                        col_hbm.at[pl.ds(blk(j + 1), IB)],
                        idxc[Y], isem).wait()
                    nxt_idx = idxc[Y].at[0]
                else:
                    nxt_idx = idxc[X].at[b + 1]
                pltpu.async_copy(xp_hbm.at[nxt_idx], rows[nrb], gsem)
                pltpu.make_async_copy(
                    xp_hbm.at[idxc[X].at[b]], rows[rb], gsem).wait()

                for gg in range(KCH // 16):
                    rvec = idxr[X][b, pl.ds(gg * 16, 16)]
                    cvec = idxc[X][b, pl.ds(gg * 16, 16)]
                    av = plsc.load_gather(alar_v, [rvec * 2])
                    bv = plsc.load_gather(alar_v, [cvec * 2 + 1])
                    alpha = av + bv
                    alpha = jnp.where(alpha >= 0.0, alpha, alpha * 0.2)
                    pb[X][pl.ds(b * KCH + gg * 16, 16)] = jnp.exp(alpha)

                def _scale(e, _, _rows=rows[rb], _p=pb[X], _off=b * KCH):
                    pv = plsc.load_gather(
                        _p, [jnp.zeros((16,), jnp.int32) + (_off + e)])
                    for jv in range(8):
                        _rows[e, pl.ds(jv * 16, 16)] = (
                            _rows[e, pl.ds(jv * 16, 16)] * pv)
                    return 0
                lax.fori_loop(0, KCH, _scale, 0, unroll=4)

                pltpu.sync_copy(rows[rb], u_sh.at[idxr[X].at[b]], add=True)
                if b % 2 == 1:
                    for gg in range(2 * KCH // 16):
                        sidx_v[pl.ds(gg * 16, 16)] = idxr[X][
                            b - 1 + gg // (KCH // 16),
                            pl.ds((gg % (KCH // 16)) * 16, 16)]
                    pltpu.sync_copy(
                        pb[X].at[pl.ds((b - 1) * KCH, 2 * KCH)],
                        s_sh.at[sidx_v], add=True)
            pltpu.async_copy(row_hbm.at[pl.ds(blk(j + 2), IB)],
                             idxr[X], isem)
            pltpu.async_copy(col_hbm.at[pl.ds(blk(j + 2), IB)],
                             idxc[X], isem)
        return 0

    lax.fori_loop(0, nblk // 2, _super, 0)

    pltpu.make_async_copy(xp_hbm.at[idxcA.at[0]], rows0, gsem).wait()
    pltpu.make_async_copy(
        row_hbm.at[pl.ds(blk(nblk + 1), IB)], idxrB, isem).wait()
    pltpu.make_async_copy(
        col_hbm.at[pl.ds(blk(nblk + 1), IB)], idxcB, isem).wait()

    plsc.subcore_barrier()

    pltpu.sync_copy(u_sh.at[pl.ds(rbase, ROWS_PER_TILE)],
                    u_hbm.at[pl.ds(c * NP + rbase, ROWS_PER_TILE)])
    pltpu.sync_copy(s_sh.at[pl.ds(sbase, 640)],
                    s_hbm.at[pl.ds(c * NP + sbase, 640)])


def _aggregate(alar_flat, row_p, col_p, xp):
    mesh = plsc.VectorSubcoreMesh(core_axis_name="c", subcore_axis_name="s")
    return pl.kernel(
        _sc_body,
        out_type=[
            jax.ShapeDtypeStruct((2 * NP, C), jnp.float32),
            jax.ShapeDtypeStruct((2 * NP,), jnp.float32),
        ],
        mesh=mesh,
        scratch_types=[
            pltpu.VMEM((2 * NP,), jnp.float32),
            pltpu.VMEM((IB, KCH), jnp.int32),
            pltpu.VMEM((IB, KCH), jnp.int32),
            pltpu.VMEM((IB, KCH), jnp.int32),
            pltpu.VMEM((IB, KCH), jnp.int32),
            pltpu.VMEM((KCH, C), jnp.float32),
            pltpu.VMEM((KCH, C), jnp.float32),
            pltpu.VMEM((IB * KCH,), jnp.float32),
            pltpu.VMEM((IB * KCH,), jnp.float32),
            pltpu.VMEM((2 * KCH,), jnp.int32),
            pltpu.VMEM((640,), jnp.float32),
            pltpu.VMEM_SHARED((NPA, C), jnp.float32),
            pltpu.VMEM_SHARED((NP,), jnp.float32),
            pltpu.SemaphoreType.DMA,
            pltpu.SemaphoreType.DMA,
        ],
        compiler_params=pltpu.CompilerParams(needs_layout_passes=False),
    )(alar_flat, row_p, col_p, xp)


def _fin_body(u0_ref, u1_ref, s0_ref, s1_ref, x_ref, b_ref, g_ref, be_ref,
              o_ref):
    u = u0_ref[...] + u1_ref[...]
    ssum = s0_ref[...] + s1_ref[...] + 1e-16
    out = u / ssum + b_ref[...]
    mu = jnp.mean(out, axis=-1, keepdims=True)
    d = out - mu
    var = jnp.mean(d * d, axis=-1, keepdims=True)
    h = d * lax.rsqrt(var + 1e-5) * g_ref[...] + be_ref[...]
    o_ref[...] = jnp.maximum(h, 0.0) + x_ref[...]


def _finalize(u_flat, s_flat, x, bias, ln_gamma, ln_beta):
    bn = 80
    off = NP // bn
    return pl.pallas_call(
        _fin_body,
        grid=(N // bn,),
        in_specs=[
            pl.BlockSpec((bn, C), lambda b: (b, 0)),
            pl.BlockSpec((bn, C), lambda b: (b + off, 0)),
            pl.BlockSpec((bn, 1), lambda b: (b, 0)),
            pl.BlockSpec((bn, 1), lambda b: (b + off, 0)),
            pl.BlockSpec((bn, C), lambda b: (b, 0)),
            pl.BlockSpec((C,), lambda b: (0,)),
            pl.BlockSpec((C,), lambda b: (0,)),
            pl.BlockSpec((C,), lambda b: (0,)),
        ],
        out_specs=pl.BlockSpec((bn, C), lambda b: (b, 0)),
        out_shape=jax.ShapeDtypeStruct((N, C), jnp.float32),
    )(u_flat, u_flat, s_flat.reshape(2 * NP, 1), s_flat.reshape(2 * NP, 1),
      x, bias, ln_gamma, ln_beta)


def kernel(x, edge_index, W, att, bias, ln_gamma, ln_beta):
    x_pad = jnp.concatenate(
        [x, jnp.zeros((NP - N, C), dtype=jnp.float32)], axis=0)
    att2 = jnp.stack([att[0, 0, :C], att[0, 0, C:]], axis=1)

    xp, alar = _project(x_pad, W, att2)
    alar_flat = alar.reshape(2 * NP)

    nrows = TILES * NCHUNK + 2 * IB
    pad_idx = jnp.full((nrows * KCH - E,), N, dtype=jnp.int32)
    row_p = jnp.concatenate([edge_index[0], pad_idx]).reshape(nrows, KCH)
    col_p = jnp.concatenate([edge_index[1], pad_idx]).reshape(nrows, KCH)

    u_flat, s_flat = _aggregate(alar_flat, row_p, col_p, xp)

    return _finalize(u_flat, s_flat, x, bias, ln_gamma, ln_beta)


# speedup vs baseline: 1.1418x; 1.1418x over previous
import functools

import jax
import jax.numpy as jnp
from jax import lax
from jax.experimental import pallas as pl
from jax.experimental.pallas import tpu as pltpu
from jax.experimental.pallas import tpu_sc as plsc

N = 10000
E = 160000
C = 128

NP = 10240
NPA = 10112
EP = 163840
TILES = 32
E_TILE = EP // TILES
KCH = 64
NCHUNK = E_TILE // KCH
IB = 4
NBLK = NCHUNK // IB
NBLK_SLOW = 28
NBLK_FAST = 12
ROWS_PER_TILE = NPA // 16


def _mm_body(x_ref, w_ref, a2_ref, xp_ref, alar_ref):
    xb = x_ref[...]
    xp = jnp.dot(xb, w_ref[...], preferred_element_type=jnp.float32)
    xp_ref[...] = xp
    alar_ref[...] = jnp.dot(xp, a2_ref[...], preferred_element_type=jnp.float32)


def _project(x_pad, W, att2):
    return pl.pallas_call(
        _mm_body,
        grid=(NP // 512,),
        in_specs=[
            pl.BlockSpec((512, C), lambda b: (b, 0)),
            pl.BlockSpec((C, C), lambda b: (0, 0)),
            pl.BlockSpec((C, 2), lambda b: (0, 0)),
        ],
        out_specs=[
            pl.BlockSpec((512, C), lambda b: (b, 0)),
            pl.BlockSpec((512, 2), lambda b: (b, 0)),
        ],
        out_shape=[
            jax.ShapeDtypeStruct((NP, C), jnp.float32),
            jax.ShapeDtypeStruct((NP, 2), jnp.float32),
        ],
    )(x_pad, W, att2)


def _sc_body(alar_hbm, row_hbm, col_hbm, xp_hbm, u_hbm, s_hbm,
             alar_v, idxrA, idxrB, idxcA, idxcB, rows0, rows1, pA, pB,
             sidx_v, zrow_v, u_sh, s_sh, gsem, isem):
    idxr = (idxrA, idxrB)
    idxc = (idxcA, idxcB)
    rows = (rows0, rows1)
    pb = (pA, pB)

    c = lax.axis_index("c")
    s = lax.axis_index("s")
    rbase = s * ROWS_PER_TILE
    nblk = jnp.where(c == 0, NBLK_SLOW, NBLK_FAST)
    cbase = jnp.where(c == 0, s * (NBLK_SLOW * IB),
                      16 * (NBLK_SLOW * IB) + s * (NBLK_FAST * IB))

    def blk(j):
        return cbase + j * IB

    zero16 = jnp.zeros((16,), jnp.float32)

    def _zrows(i, _):
        r = i // 8
        j = i - r * 8
        rows0[r, pl.ds(j * 16, 16)] = zero16
        return 0
    lax.fori_loop(0, KCH * 8, _zrows, 0)

    def _zline(i, _):
        zrow_v[pl.ds(i * 16, 16)] = zero16
        return 0
    lax.fori_loop(0, 640 // 16, _zline, 0)

    sbase = s * 640

    for k in range(ROWS_PER_TILE // KCH):
        pltpu.sync_copy(rows0, u_sh.at[pl.ds(rbase + k * KCH, KCH)])
    rem = ROWS_PER_TILE - (ROWS_PER_TILE // KCH) * KCH
    pltpu.sync_copy(
        rows0.at[pl.ds(0, rem)],
        u_sh.at[pl.ds(rbase + ROWS_PER_TILE - rem, rem)])
    pltpu.sync_copy(zrow_v, s_sh.at[pl.ds(sbase, 640)])

    pltpu.sync_copy(alar_hbm, alar_v)

    plsc.subcore_barrier()

    pltpu.sync_copy(row_hbm.at[pl.ds(blk(0), IB)], idxrA)
    pltpu.sync_copy(col_hbm.at[pl.ds(blk(0), IB)], idxcA)
    pltpu.async_copy(row_hbm.at[pl.ds(blk(1), IB)], idxrB, isem)
    pltpu.async_copy(col_hbm.at[pl.ds(blk(1), IB)], idxcB, isem)
    pltpu.async_copy(xp_hbm.at[idxcA.at[0]], rows0, gsem)

    def _super(g2, _):
        for jj in (0, 1):
            j = 2 * g2 + jj
            X = jj
            Y = 1 - jj
            for b in range(IB):
                rb = b % 2
                nrb = 1 - rb
                if b == IB - 1:
                    pltpu.make_async_copy(
                        row_hbm.at[pl.ds(blk(j + 1), IB)],
                        idxr[Y], isem).wait()
                    pltpu.make_async_copy(
                        col_hbm.at[pl.ds(blk(j + 1), IB)],
                        idxc[Y], isem).wait()
                    nxt_idx = idxc[Y].at[0]
                else:
                    nxt_idx = idxc[X].at[b + 1]
                pltpu.async_copy(xp_hbm.at[nxt_idx], rows[nrb], gsem)
                pltpu.make_async_copy(
                    xp_hbm.at[idxc[X].at[b]], rows[rb], gsem).wait()

                for gg in range(KCH // 16):
                    rvec = idxr[X][b, pl.ds(gg * 16, 16)]
                    cvec = idxc[X][b, pl.ds(gg * 16, 16)]
                    av = plsc.load_gather(alar_v, [rvec * 2])
                    bv = plsc.load_gather(alar_v, [cvec * 2 + 1])
                    alpha = av + bv
                    alpha = jnp.where(alpha >= 0.0, alpha, alpha * 0.2)
                    pb[X][pl.ds(b * KCH + gg * 16, 16)] = jnp.exp(alpha)

                def _scale(e, _, _rows=rows[rb], _p=pb[X], _off=b * KCH):
                    pv = plsc.load_gather(
                        _p, [jnp.zeros((16,), jnp.int32) + (_off + e)])
                    for jv in range(8):
                        _rows[e, pl.ds(jv * 16, 16)] = (
                            _rows[e, pl.ds(jv * 16, 16)] * pv)
                    return 0
                lax.fori_loop(0, KCH, _scale, 0, unroll=4)

                pltpu.sync_copy(rows[rb], u_sh.at[idxr[X].at[b]], add=True)
                if b % 2 == 1:
                    for gg in range(2 * KCH // 16):
                        sidx_v[pl.ds(gg * 16, 16)] = idxr[X][
                            b - 1 + gg // (KCH // 16),
                            pl.ds((gg % (KCH // 16)) * 16, 16)]
                    pltpu.sync_copy(
                        pb[X].at[pl.ds((b - 1) * KCH, 2 * KCH)],
                        s_sh.at[sidx_v], add=True)
            pltpu.async_copy(row_hbm.at[pl.ds(blk(j + 2), IB)],
                             idxr[X], isem)
            pltpu.async_copy(col_hbm.at[pl.ds(blk(j + 2), IB)],
                             idxc[X], isem)
        return 0

    lax.fori_loop(0, nblk // 2, _super, 0)

    pltpu.make_async_copy(xp_hbm.at[idxcA.at[0]], rows0, gsem).wait()
    pltpu.make_async_copy(
        row_hbm.at[pl.ds(blk(nblk + 1), IB)], idxrB, isem).wait()
    pltpu.make_async_copy(
        col_hbm.at[pl.ds(blk(nblk + 1), IB)], idxcB, isem).wait()

    plsc.subcore_barrier()

    pltpu.sync_copy(u_sh.at[pl.ds(rbase, ROWS_PER_TILE)],
                    u_hbm.at[pl.ds(c * NP + rbase, ROWS_PER_TILE)])
    pltpu.sync_copy(s_sh.at[pl.ds(sbase, 640)],
                    s_hbm.at[pl.ds(c * NP + sbase, 640)])


def _aggregate(alar_flat, row_p, col_p, xp):
    mesh = plsc.VectorSubcoreMesh(core_axis_name="c", subcore_axis_name="s")
    return pl.kernel(
        _sc_body,
        out_type=[
            jax.ShapeDtypeStruct((2 * NP, C), jnp.float32),
            jax.ShapeDtypeStruct((2 * NP,), jnp.float32),
        ],
        mesh=mesh,
        scratch_types=[
            pltpu.VMEM((2 * NP,), jnp.float32),
            pltpu.VMEM((IB, KCH), jnp.int32),
            pltpu.VMEM((IB, KCH), jnp.int32),
            pltpu.VMEM((IB, KCH), jnp.int32),
            pltpu.VMEM((IB, KCH), jnp.int32),
            pltpu.VMEM((KCH, C), jnp.float32),
            pltpu.VMEM((KCH, C), jnp.float32),
            pltpu.VMEM((IB * KCH,), jnp.float32),
            pltpu.VMEM((IB * KCH,), jnp.float32),
            pltpu.VMEM((2 * KCH,), jnp.int32),
            pltpu.VMEM((640,), jnp.float32),
            pltpu.VMEM_SHARED((NPA, C), jnp.float32),
            pltpu.VMEM_SHARED((NP,), jnp.float32),
            pltpu.SemaphoreType.DMA,
            pltpu.SemaphoreType.DMA,
        ],
        compiler_params=pltpu.CompilerParams(needs_layout_passes=False),
    )(alar_flat, row_p, col_p, xp)


def _fin_body(u0_ref, u1_ref, s0_ref, s1_ref, x_ref, b_ref, g_ref, be_ref,
              o_ref):
    u = u0_ref[...] + u1_ref[...]
    ssum = s0_ref[...] + s1_ref[...] + 1e-16
    out = u / ssum + b_ref[...]
    mu = jnp.mean(out, axis=-1, keepdims=True)
    d = out - mu
    var = jnp.mean(d * d, axis=-1, keepdims=True)
    h = d * lax.rsqrt(var + 1e-5) * g_ref[...] + be_ref[...]
    o_ref[...] = jnp.maximum(h, 0.0) + x_ref[...]


def _finalize(u_flat, s_flat, x, bias, ln_gamma, ln_beta):
    bn = 80
    off = NP // bn
    return pl.pallas_call(
        _fin_body,
        grid=(N // bn,),
        in_specs=[
            pl.BlockSpec((bn, C), lambda b: (b, 0)),
            pl.BlockSpec((bn, C), lambda b: (b + off, 0)),
            pl.BlockSpec((bn, 1), lambda b: (b, 0)),
            pl.BlockSpec((bn, 1), lambda b: (b + off, 0)),
            pl.BlockSpec((bn, C), lambda b: (b, 0)),
            pl.BlockSpec((C,), lambda b: (0,)),
            pl.BlockSpec((C,), lambda b: (0,)),
            pl.BlockSpec((C,), lambda b: (0,)),
        ],
        out_specs=pl.BlockSpec((bn, C), lambda b: (b, 0)),
        out_shape=jax.ShapeDtypeStruct((N, C), jnp.float32),
    )(u_flat, u_flat, s_flat.reshape(2 * NP, 1), s_flat.reshape(2 * NP, 1),
      x, bias, ln_gamma, ln_beta)


def kernel(x, edge_index, W, att, bias, ln_gamma, ln_beta):
    x_pad = jnp.concatenate(
        [x, jnp.zeros((NP - N, C), dtype=jnp.float32)], axis=0)
    att2 = jnp.stack([att[0, 0, :C], att[0, 0, C:]], axis=1)

    xp, alar = _project(x_pad, W, att2)
    alar_flat = alar.reshape(2 * NP)

    nrows = TILES * NCHUNK + 2 * IB
    pad_idx = jnp.full((nrows * KCH - E,), N, dtype=jnp.int32)
    row_p = jnp.concatenate([edge_index[0], pad_idx]).reshape(nrows, KCH)
    col_p = jnp.concatenate([edge_index[1], pad_idx]).reshape(nrows, KCH)

    u_flat, s_flat = _aggregate(alar_flat, row_p, col_p, xp)

    return _finalize(u_flat, s_flat, x, bias, ln_gamma, ln_beta)
